# baseline (speedup 1.0000x reference)
import functools

import jax
import jax.numpy as jnp
from jax import lax
from jax.experimental import pallas as pl
from jax.experimental.pallas import tpu as pltpu
from jax.experimental.pallas import tpu_sc as plsc

_N = 10000
_D = 128
_G = 64
_NCLS = 10

_LANES = 16
_NSUB = 16
_NCORE = 2
_NW = _NSUB * _NCORE

_CHUNK = 128
_ACC_ROWS = 10240
_TRASH = _N
_ZROWS = _ACC_ROWS // _NSUB

_ROWBLK = 2000

_K = 1


def _fix_dst(src2d, dst2d):
    chunks = src2d.shape[0]

    def body(s_ref, d_ref, o_ref):
        o_ref[...] = jnp.where(s_ref[...] == d_ref[...], _TRASH, d_ref[...])

    return pl.pallas_call(
        body,
        out_shape=jax.ShapeDtypeStruct((chunks, _CHUNK), jnp.int32),
    )(src2d, dst2d)


def _sc_propagate(xl, src2d, dst2d, zeros_init):
    chunks = src2d.shape[0]
    cpt = chunks // _NW
    mesh = plsc.VectorSubcoreMesh(core_axis_name="c", subcore_axis_name="s")

    @functools.partial(
        pl.kernel,
        out_type=jax.ShapeDtypeStruct((_NCORE * _ACC_ROWS, _D), jnp.float32),
        mesh=mesh,
        scratch_types=[
            pltpu.VMEM((cpt, _CHUNK), jnp.int32),
            pltpu.VMEM((cpt, _CHUNK), jnp.int32),
            pltpu.VMEM((_CHUNK, _D), jnp.float32),
            pltpu.SemaphoreType.DMA,
            pltpu.VMEM_SHARED((_ACC_ROWS, _D), jnp.float32),
        ],
    )
    def k(xl_hbm, src_hbm, dst_hbm, z_hbm, out_hbm, src_i, dst_i, rows_v,
          gsem, acc):
        c = lax.axis_index("c")
        s = lax.axis_index("s")
        pltpu.sync_copy(z_hbm, acc.at[pl.ds(s * _ZROWS, _ZROWS)])
        plsc.subcore_barrier()
        base = (c * _NSUB + s) * cpt
        pltpu.sync_copy(src_hbm.at[pl.ds(base, cpt)], src_i)
        pltpu.sync_copy(dst_hbm.at[pl.ds(base, cpt)], dst_i)

        def body(j, carry):
            pltpu.async_copy(xl_hbm.at[src_i.at[j]], rows_v, gsem).wait()
            pltpu.sync_copy(rows_v, acc.at[dst_i.at[j]], add=True)
            return carry

        lax.fori_loop(0, cpt, body, 0)
        plsc.subcore_barrier()
        pltpu.sync_copy(
            acc.at[pl.ds(s * _ZROWS, _ZROWS)],
            out_hbm.at[pl.ds(c * _ACC_ROWS + s * _ZROWS, _ZROWS)])

    return k(xl, src2d, dst2d, zeros_init)


def _first_lin(x, wl, bl):
    def body(x_ref, w_ref, b_ref, o_ref):
        o_ref[...] = (
            jnp.dot(x_ref[...], w_ref[...], preferred_element_type=jnp.float32)
            + b_ref[...])

    return pl.pallas_call(
        body,
        grid=(_N // _ROWBLK,),
        in_specs=[
            pl.BlockSpec((_ROWBLK, _D), lambda i: (i, 0)),
            pl.BlockSpec((_D, _D), lambda i: (0, 0)),
            pl.BlockSpec((1, _D), lambda i: (0, 0)),
        ],
        out_specs=pl.BlockSpec((_ROWBLK, _D), lambda i: (i, 0)),
        out_shape=jax.ShapeDtypeStruct((_N, _D), jnp.float32),
    )(x, wl, bl)


def _mid_layer(h, p0, p1, ws, bs, g, be, wl, bl):
    def body(h_ref, p0_ref, p1_ref, ws_ref, bs_ref, g_ref, be_ref,
             wl_ref, bl_ref, ho_ref, xo_ref):
        z = (jnp.dot(h_ref[...], ws_ref[...], preferred_element_type=jnp.float32)
             + bs_ref[...] + p0_ref[...] + p1_ref[...])
        z = jnp.maximum(z, 0.0)
        m = jnp.mean(z, axis=-1, keepdims=True)
        v = jnp.mean((z - m) ** 2, axis=-1, keepdims=True)
        zn = (z - m) * lax.rsqrt(v + 1e-5) * g_ref[...] + be_ref[...]
        ho_ref[...] = zn
        xo_ref[...] = (
            jnp.dot(zn, wl_ref[...], preferred_element_type=jnp.float32)
            + bl_ref[...])

    row = pl.BlockSpec((_ROWBLK, _D), lambda i: (i, 0))
    full = pl.BlockSpec((_D, _D), lambda i: (0, 0))
    vec = pl.BlockSpec((1, _D), lambda i: (0, 0))
    return pl.pallas_call(
        body,
        grid=(_N // _ROWBLK,),
        in_specs=[row, row, row, full, vec, vec, vec, full, vec],
        out_specs=[row, row],
        out_shape=[jax.ShapeDtypeStruct((_N, _D), jnp.float32),
                   jax.ShapeDtypeStruct((_N, _D), jnp.float32)],
    )(h, p0, p1, ws, bs, g, be, wl, bl)


def _final_stage(h, p0, p1, ws, bs, batch2d, w1, b1, w2p, b2p):
    steps = _N // _ROWBLK

    def body(h_ref, p0_ref, p1_ref, ws_ref, bs_ref, bt_ref, w1_ref, b1_ref,
             w2_ref, b2_ref, o_ref, pool_acc, cnt_acc):
        i = pl.program_id(0)

        @pl.when(i == 0)
        def _():
            pool_acc[...] = jnp.zeros_like(pool_acc)
            cnt_acc[...] = jnp.zeros_like(cnt_acc)

        z = (jnp.dot(h_ref[...], ws_ref[...], preferred_element_type=jnp.float32)
             + bs_ref[...] + p0_ref[...] + p1_ref[...])
        z = jnp.maximum(z, 0.0)
        onehot = (bt_ref[...] ==
                  lax.broadcasted_iota(jnp.int32, (_ROWBLK, _G), 1)
                  ).astype(jnp.float32)
        dn = (((0,), (0,)), ((), ()))
        pool_acc[...] += lax.dot_general(
            onehot, z, dn, preferred_element_type=jnp.float32)
        cnt_acc[...] += lax.dot_general(
            onehot, jnp.ones((_ROWBLK, _D), jnp.float32), dn,
            preferred_element_type=jnp.float32)

        @pl.when(i == steps - 1)
        def _():
            pooled = pool_acc[...] / jnp.maximum(cnt_acc[...], 1.0)
            z1 = (jnp.dot(pooled, w1_ref[...],
                          preferred_element_type=jnp.float32) + b1_ref[...])
            logits = (jnp.dot(z1, w2_ref[...],
                              preferred_element_type=jnp.float32) + b2_ref[...])
            colmask = lax.broadcasted_iota(jnp.int32, (_G, _D), 1) < _NCLS
            mlog = jnp.where(colmask, logits, -1e30)
            mx = jnp.max(mlog, axis=1, keepdims=True)
            ex = jnp.where(colmask, jnp.exp(logits - mx), 0.0)
            lse = jnp.log(jnp.sum(ex, axis=1, keepdims=True)) + mx
            o_ref[...] = logits - lse

    row = pl.BlockSpec((_ROWBLK, _D), lambda i: (i, 0))
    full = pl.BlockSpec((_D, _D), lambda i: (0, 0))
    vec = pl.BlockSpec((1, _D), lambda i: (0, 0))
    return pl.pallas_call(
        body,
        grid=(steps,),
        in_specs=[row, row, row, full, vec,
                  pl.BlockSpec((_ROWBLK, 1), lambda i: (i, 0)),
                  full, vec, full, vec],
        out_specs=pl.BlockSpec((_G, _D), lambda i: (0, 0)),
        out_shape=jax.ShapeDtypeStruct((_G, _D), jnp.float32),
        scratch_shapes=[pltpu.VMEM((_G, _D), jnp.float32),
                        pltpu.VMEM((_G, _D), jnp.float32)],
    )(h, p0, p1, ws, bs, batch2d, w1, b1, w2p, b2p)


def kernel(x, edge_index, batch, Wl0, bl0, Ws0, bs0, Wl1, bl1, Ws1, bs1,
           Wl2, bl2, Ws2, bs2, g0, be0, g1, be1, W1, b1, W2, b2):
    src = edge_index[0]
    dst = edge_index[1]
    e = src.shape[0]
    chunks = -(-e // _CHUNK)
    chunks = -(-chunks // (_NW * 8)) * (_NW * 8)
    pad = chunks * _CHUNK - e
    src2d = jnp.concatenate([src, jnp.zeros((pad,), jnp.int32)]).reshape(
        chunks, _CHUNK)
    dst2d = jnp.concatenate([dst, jnp.zeros((pad,), jnp.int32)]).reshape(
        chunks, _CHUNK)
    dst2d = _fix_dst(src2d, dst2d)
    zeros_init = jnp.zeros((_ZROWS, _D), jnp.float32)
    batch2d = batch.reshape(_N, 1)

    r = lambda v: v.reshape(1, _D)
    w2p = jnp.concatenate([W2, jnp.zeros((_D, _D - _NCLS), jnp.float32)], axis=1)
    b2p = jnp.concatenate([b2, jnp.zeros((_D - _NCLS,), jnp.float32)]).reshape(1, _D)

    xl0 = _first_lin(x, Wl0, r(bl0))
    parts = _sc_propagate(xl0, src2d, dst2d, zeros_init)
    h1, xl1 = _mid_layer(x, parts[:_N], parts[_ACC_ROWS:_ACC_ROWS + _N], Ws0,
                         r(bs0), r(g0), r(be0), Wl1, r(bl1))
    parts = _sc_propagate(xl1, src2d, dst2d, zeros_init)
    h2, xl2 = _mid_layer(h1, parts[:_N], parts[_ACC_ROWS:_ACC_ROWS + _N], Ws1,
                         r(bs1), r(g1), r(be1), Wl2, r(bl2))
    parts = _sc_propagate(xl2, src2d, dst2d, zeros_init)
    out = _final_stage(h2, parts[:_N], parts[_ACC_ROWS:_ACC_ROWS + _N], Ws2,
                       r(bs2), batch2d, W1, r(b1), w2p, b2p)
    return out[:, :_NCLS]


# speedup vs baseline: 1.4665x; 1.4665x over previous
import functools

import jax
import jax.numpy as jnp
from jax import lax
from jax.experimental import pallas as pl
from jax.experimental.pallas import tpu as pltpu
from jax.experimental.pallas import tpu_sc as plsc

_N = 10000
_D = 128
_G = 64
_NCLS = 10

_LANES = 16
_NSUB = 16
_NCORE = 2
_NW = _NSUB * _NCORE

_CHUNK = 128
_ACC_ROWS = 10240
_TRASH = _N
_ZROWS = _ACC_ROWS // _NSUB

_ROWBLK = 2000

_K = 1


def _fix_dst(src2d, dst2d):
    chunks = src2d.shape[0]

    def body(s_ref, d_ref, o_ref):
        o_ref[...] = jnp.where(s_ref[...] == d_ref[...], _TRASH, d_ref[...])

    return pl.pallas_call(
        body,
        out_shape=jax.ShapeDtypeStruct((chunks, _CHUNK), jnp.int32),
    )(src2d, dst2d)


_CPT0_FRAC = 0.41


def _sc_propagate(xl, src2d, dst2d, zeros_init):
    chunks = src2d.shape[0]
    per_pair = chunks // _NSUB
    cpt0 = int(per_pair * _CPT0_FRAC)
    cpt1 = per_pair - cpt0
    mesh = plsc.VectorSubcoreMesh(core_axis_name="c", subcore_axis_name="s")

    @functools.partial(
        pl.kernel,
        out_type=jax.ShapeDtypeStruct((_NCORE * _ACC_ROWS, _D), jnp.float32),
        mesh=mesh,
        scratch_types=[
            pltpu.VMEM((_CHUNK,), jnp.int32),
            pltpu.VMEM((_CHUNK,), jnp.int32),
            pltpu.VMEM((_CHUNK, _D), jnp.float32),
            pltpu.SemaphoreType.DMA,
            pltpu.VMEM_SHARED((_ACC_ROWS, _D), jnp.float32),
        ],
    )
    def k(xl_hbm, src_hbm, dst_hbm, z_hbm, out_hbm, src_v, dst_v, rows_v,
          gsem, acc):
        c = lax.axis_index("c")
        s = lax.axis_index("s")
        pltpu.sync_copy(z_hbm, acc.at[pl.ds(s * _ZROWS, _ZROWS)])
        plsc.subcore_barrier()
        base = jnp.where(c == 0, s * cpt0, _NSUB * cpt0 + s * cpt1)
        trip = jnp.where(c == 0, cpt0, cpt1)

        def body(j, carry):
            row = base + j
            pltpu.sync_copy(src_hbm.at[row], src_v)
            pltpu.sync_copy(dst_hbm.at[row], dst_v)
            pltpu.async_copy(xl_hbm.at[src_v], rows_v, gsem).wait()
            pltpu.sync_copy(rows_v, acc.at[dst_v], add=True)
            return carry

        lax.fori_loop(0, trip, body, 0)
        plsc.subcore_barrier()
        pltpu.sync_copy(
            acc.at[pl.ds(s * _ZROWS, _ZROWS)],
            out_hbm.at[pl.ds(c * _ACC_ROWS + s * _ZROWS, _ZROWS)])

    return k(xl, src2d, dst2d, zeros_init)


def _first_lin(x, wl, bl):
    def body(x_ref, w_ref, b_ref, o_ref):
        o_ref[...] = (
            jnp.dot(x_ref[...], w_ref[...], preferred_element_type=jnp.float32)
            + b_ref[...])

    return pl.pallas_call(
        body,
        grid=(_N // _ROWBLK,),
        in_specs=[
            pl.BlockSpec((_ROWBLK, _D), lambda i: (i, 0)),
            pl.BlockSpec((_D, _D), lambda i: (0, 0)),
            pl.BlockSpec((1, _D), lambda i: (0, 0)),
        ],
        out_specs=pl.BlockSpec((_ROWBLK, _D), lambda i: (i, 0)),
        out_shape=jax.ShapeDtypeStruct((_N, _D), jnp.float32),
    )(x, wl, bl)


def _mid_layer(h, p0, p1, ws, bs, g, be, wl, bl):
    def body(h_ref, p0_ref, p1_ref, ws_ref, bs_ref, g_ref, be_ref,
             wl_ref, bl_ref, ho_ref, xo_ref):
        z = (jnp.dot(h_ref[...], ws_ref[...], preferred_element_type=jnp.float32)
             + bs_ref[...] + p0_ref[...] + p1_ref[...])
        z = jnp.maximum(z, 0.0)
        m = jnp.mean(z, axis=-1, keepdims=True)
        v = jnp.mean((z - m) ** 2, axis=-1, keepdims=True)
        zn = (z - m) * lax.rsqrt(v + 1e-5) * g_ref[...] + be_ref[...]
        ho_ref[...] = zn
        xo_ref[...] = (
            jnp.dot(zn, wl_ref[...], preferred_element_type=jnp.float32)
            + bl_ref[...])

    row = pl.BlockSpec((_ROWBLK, _D), lambda i: (i, 0))
    full = pl.BlockSpec((_D, _D), lambda i: (0, 0))
    vec = pl.BlockSpec((1, _D), lambda i: (0, 0))
    return pl.pallas_call(
        body,
        grid=(_N // _ROWBLK,),
        in_specs=[row, row, row, full, vec, vec, vec, full, vec],
        out_specs=[row, row],
        out_shape=[jax.ShapeDtypeStruct((_N, _D), jnp.float32),
                   jax.ShapeDtypeStruct((_N, _D), jnp.float32)],
    )(h, p0, p1, ws, bs, g, be, wl, bl)


def _final_stage(h, p0, p1, ws, bs, batch2d, w1, b1, w2p, b2p):
    steps = _N // _ROWBLK

    def body(h_ref, p0_ref, p1_ref, ws_ref, bs_ref, bt_ref, w1_ref, b1_ref,
             w2_ref, b2_ref, o_ref, pool_acc, cnt_acc):
        i = pl.program_id(0)

        @pl.when(i == 0)
        def _():
            pool_acc[...] = jnp.zeros_like(pool_acc)
            cnt_acc[...] = jnp.zeros_like(cnt_acc)

        z = (jnp.dot(h_ref[...], ws_ref[...], preferred_element_type=jnp.float32)
             + bs_ref[...] + p0_ref[...] + p1_ref[...])
        z = jnp.maximum(z, 0.0)
        onehot = (bt_ref[...] ==
                  lax.broadcasted_iota(jnp.int32, (_ROWBLK, _G), 1)
                  ).astype(jnp.float32)
        dn = (((0,), (0,)), ((), ()))
        pool_acc[...] += lax.dot_general(
            onehot, z, dn, preferred_element_type=jnp.float32)
        cnt_acc[...] += lax.dot_general(
            onehot, jnp.ones((_ROWBLK, _D), jnp.float32), dn,
            preferred_element_type=jnp.float32)

        @pl.when(i == steps - 1)
        def _():
            pooled = pool_acc[...] / jnp.maximum(cnt_acc[...], 1.0)
            z1 = (jnp.dot(pooled, w1_ref[...],
                          preferred_element_type=jnp.float32) + b1_ref[...])
            logits = (jnp.dot(z1, w2_ref[...],
                              preferred_element_type=jnp.float32) + b2_ref[...])
            colmask = lax.broadcasted_iota(jnp.int32, (_G, _D), 1) < _NCLS
            mlog = jnp.where(colmask, logits, -1e30)
            mx = jnp.max(mlog, axis=1, keepdims=True)
            ex = jnp.where(colmask, jnp.exp(logits - mx), 0.0)
            lse = jnp.log(jnp.sum(ex, axis=1, keepdims=True)) + mx
            o_ref[...] = logits - lse

    row = pl.BlockSpec((_ROWBLK, _D), lambda i: (i, 0))
    full = pl.BlockSpec((_D, _D), lambda i: (0, 0))
    vec = pl.BlockSpec((1, _D), lambda i: (0, 0))
    return pl.pallas_call(
        body,
        grid=(steps,),
        in_specs=[row, row, row, full, vec,
                  pl.BlockSpec((_ROWBLK, 1), lambda i: (i, 0)),
                  full, vec, full, vec],
        out_specs=pl.BlockSpec((_G, _D), lambda i: (0, 0)),
        out_shape=jax.ShapeDtypeStruct((_G, _D), jnp.float32),
        scratch_shapes=[pltpu.VMEM((_G, _D), jnp.float32),
                        pltpu.VMEM((_G, _D), jnp.float32)],
    )(h, p0, p1, ws, bs, batch2d, w1, b1, w2p, b2p)


def kernel(x, edge_index, batch, Wl0, bl0, Ws0, bs0, Wl1, bl1, Ws1, bs1,
           Wl2, bl2, Ws2, bs2, g0, be0, g1, be1, W1, b1, W2, b2):
    src = edge_index[0]
    dst = edge_index[1]
    e = src.shape[0]
    chunks = -(-e // _CHUNK)
    chunks = -(-chunks // _NSUB) * _NSUB
    pad = chunks * _CHUNK - e
    src2d = jnp.concatenate([src, jnp.zeros((pad,), jnp.int32)]).reshape(
        chunks, _CHUNK)
    dst2d = jnp.concatenate([dst, jnp.zeros((pad,), jnp.int32)]).reshape(
        chunks, _CHUNK)
    dst2d = _fix_dst(src2d, dst2d)
    zeros_init = jnp.zeros((_ZROWS, _D), jnp.float32)
    batch2d = batch.reshape(_N, 1)

    r = lambda v: v.reshape(1, _D)
    w2p = jnp.concatenate([W2, jnp.zeros((_D, _D - _NCLS), jnp.float32)], axis=1)
    b2p = jnp.concatenate([b2, jnp.zeros((_D - _NCLS,), jnp.float32)]).reshape(1, _D)

    xl0 = _first_lin(x, Wl0, r(bl0))
    parts = _sc_propagate(xl0, src2d, dst2d, zeros_init)
    h1, xl1 = _mid_layer(x, parts[:_N], parts[_ACC_ROWS:_ACC_ROWS + _N], Ws0,
                         r(bs0), r(g0), r(be0), Wl1, r(bl1))
    parts = _sc_propagate(xl1, src2d, dst2d, zeros_init)
    h2, xl2 = _mid_layer(h1, parts[:_N], parts[_ACC_ROWS:_ACC_ROWS + _N], Ws1,
                         r(bs1), r(g1), r(be1), Wl2, r(bl2))
    parts = _sc_propagate(xl2, src2d, dst2d, zeros_init)
    out = _final_stage(h2, parts[:_N], parts[_ACC_ROWS:_ACC_ROWS + _N], Ws2,
                       r(bs2), batch2d, W1, r(b1), w2p, b2p)
    return out[:, :_NCLS]


# speedup vs baseline: 1.5233x; 1.0387x over previous
import functools

import jax
import jax.numpy as jnp
from jax import lax
from jax.experimental import pallas as pl
from jax.experimental.pallas import tpu as pltpu
from jax.experimental.pallas import tpu_sc as plsc

_N = 10000
_D = 128
_G = 64
_NCLS = 10

_LANES = 16
_NSUB = 16
_NCORE = 2
_NW = _NSUB * _NCORE

_CHUNK = 128
_ACC_ROWS = 10240
_TRASH = _N
_ZROWS = _ACC_ROWS // _NSUB

_ROWBLK = 2000

_K = 1


def _fix_dst(src2d, dst2d):
    chunks = src2d.shape[0]

    def body(s_ref, d_ref, o_ref):
        o_ref[...] = jnp.where(s_ref[...] == d_ref[...], _TRASH, d_ref[...])

    return pl.pallas_call(
        body,
        out_shape=jax.ShapeDtypeStruct((chunks, _CHUNK), jnp.int32),
    )(src2d, dst2d)


_CPT0_FRAC = 0.41


def _sc_propagate(xl, src2d, dst2d, zeros_init):
    chunks = src2d.shape[0]
    per_pair = chunks // _NSUB
    cpt0 = int(per_pair * _CPT0_FRAC)
    cpt1 = per_pair - cpt0
    mesh = plsc.VectorSubcoreMesh(core_axis_name="c", subcore_axis_name="s")

    @functools.partial(
        pl.kernel,
        out_type=jax.ShapeDtypeStruct((_NCORE * _ACC_ROWS, _D), jnp.float32),
        mesh=mesh,
        scratch_types=[
            pltpu.VMEM((_CHUNK,), jnp.int32),
            pltpu.VMEM((_CHUNK,), jnp.int32),
            pltpu.VMEM((_CHUNK, _D), jnp.float32),
            pltpu.SemaphoreType.DMA,
            pltpu.VMEM_SHARED((_ACC_ROWS, _D), jnp.float32),
        ],
    )
    def k(xl_hbm, src_hbm, dst_hbm, z_hbm, out_hbm, src_v, dst_v, rows_v,
          gsem, acc):
        c = lax.axis_index("c")
        s = lax.axis_index("s")
        pltpu.sync_copy(z_hbm, acc.at[pl.ds(s * _ZROWS, _ZROWS)])
        plsc.subcore_barrier()
        base = jnp.where(c == 0, s * cpt0, _NSUB * cpt0 + s * cpt1)
        trip = jnp.where(c == 0, cpt0, cpt1)

        def body(j, carry):
            row = base + j
            pltpu.sync_copy(src_hbm.at[row], src_v)
            pltpu.sync_copy(dst_hbm.at[row], dst_v)
            for u in range(_CHUNK // _LANES):
                sl = pl.ds(u * _LANES, _LANES)
                sv = src_v[sl]
                dv = dst_v[sl]
                dst_v[sl] = jnp.where(sv == dv, _TRASH, dv)
            pltpu.async_copy(xl_hbm.at[src_v], rows_v, gsem).wait()
            pltpu.sync_copy(rows_v, acc.at[dst_v], add=True)
            return carry

        lax.fori_loop(0, trip, body, 0)
        plsc.subcore_barrier()
        pltpu.sync_copy(
            acc.at[pl.ds(s * _ZROWS, _ZROWS)],
            out_hbm.at[pl.ds(c * _ACC_ROWS + s * _ZROWS, _ZROWS)])

    return k(xl, src2d, dst2d, zeros_init)


def _first_lin(x, wl, bl):
    def body(x_ref, w_ref, b_ref, o_ref):
        o_ref[...] = (
            jnp.dot(x_ref[...], w_ref[...], preferred_element_type=jnp.float32)
            + b_ref[...])

    return pl.pallas_call(
        body,
        grid=(_N // _ROWBLK,),
        in_specs=[
            pl.BlockSpec((_ROWBLK, _D), lambda i: (i, 0)),
            pl.BlockSpec((_D, _D), lambda i: (0, 0)),
            pl.BlockSpec((1, _D), lambda i: (0, 0)),
        ],
        out_specs=pl.BlockSpec((_ROWBLK, _D), lambda i: (i, 0)),
        out_shape=jax.ShapeDtypeStruct((_N, _D), jnp.float32),
    )(x, wl, bl)


def _mid_layer(h, p0, p1, ws, bs, g, be, wl, bl):
    def body(h_ref, p0_ref, p1_ref, ws_ref, bs_ref, g_ref, be_ref,
             wl_ref, bl_ref, ho_ref, xo_ref):
        z = (jnp.dot(h_ref[...], ws_ref[...], preferred_element_type=jnp.float32)
             + bs_ref[...] + p0_ref[...] + p1_ref[...])
        z = jnp.maximum(z, 0.0)
        m = jnp.mean(z, axis=-1, keepdims=True)
        v = jnp.mean((z - m) ** 2, axis=-1, keepdims=True)
        zn = (z - m) * lax.rsqrt(v + 1e-5) * g_ref[...] + be_ref[...]
        ho_ref[...] = zn
        xo_ref[...] = (
            jnp.dot(zn, wl_ref[...], preferred_element_type=jnp.float32)
            + bl_ref[...])

    row = pl.BlockSpec((_ROWBLK, _D), lambda i: (i, 0))
    full = pl.BlockSpec((_D, _D), lambda i: (0, 0))
    vec = pl.BlockSpec((1, _D), lambda i: (0, 0))
    return pl.pallas_call(
        body,
        grid=(_N // _ROWBLK,),
        in_specs=[row, row, row, full, vec, vec, vec, full, vec],
        out_specs=[row, row],
        out_shape=[jax.ShapeDtypeStruct((_N, _D), jnp.float32),
                   jax.ShapeDtypeStruct((_N, _D), jnp.float32)],
    )(h, p0, p1, ws, bs, g, be, wl, bl)


def _final_stage(h, p0, p1, ws, bs, batch2d, w1, b1, w2p, b2p):
    steps = _N // _ROWBLK

    def body(h_ref, p0_ref, p1_ref, ws_ref, bs_ref, bt_ref, w1_ref, b1_ref,
             w2_ref, b2_ref, o_ref, pool_acc, cnt_acc):
        i = pl.program_id(0)

        @pl.when(i == 0)
        def _():
            pool_acc[...] = jnp.zeros_like(pool_acc)
            cnt_acc[...] = jnp.zeros_like(cnt_acc)

        z = (jnp.dot(h_ref[...], ws_ref[...], preferred_element_type=jnp.float32)
             + bs_ref[...] + p0_ref[...] + p1_ref[...])
        z = jnp.maximum(z, 0.0)
        onehot = (bt_ref[...] ==
                  lax.broadcasted_iota(jnp.int32, (_ROWBLK, _G), 1)
                  ).astype(jnp.float32)
        dn = (((0,), (0,)), ((), ()))
        pool_acc[...] += lax.dot_general(
            onehot, z, dn, preferred_element_type=jnp.float32)
        cnt_acc[...] += lax.dot_general(
            onehot, jnp.ones((_ROWBLK, _D), jnp.float32), dn,
            preferred_element_type=jnp.float32)

        @pl.when(i == steps - 1)
        def _():
            pooled = pool_acc[...] / jnp.maximum(cnt_acc[...], 1.0)
            z1 = (jnp.dot(pooled, w1_ref[...],
                          preferred_element_type=jnp.float32) + b1_ref[...])
            logits = (jnp.dot(z1, w2_ref[...],
                              preferred_element_type=jnp.float32) + b2_ref[...])
            colmask = lax.broadcasted_iota(jnp.int32, (_G, _D), 1) < _NCLS
            mlog = jnp.where(colmask, logits, -1e30)
            mx = jnp.max(mlog, axis=1, keepdims=True)
            ex = jnp.where(colmask, jnp.exp(logits - mx), 0.0)
            lse = jnp.log(jnp.sum(ex, axis=1, keepdims=True)) + mx
            o_ref[...] = logits - lse

    row = pl.BlockSpec((_ROWBLK, _D), lambda i: (i, 0))
    full = pl.BlockSpec((_D, _D), lambda i: (0, 0))
    vec = pl.BlockSpec((1, _D), lambda i: (0, 0))
    return pl.pallas_call(
        body,
        grid=(steps,),
        in_specs=[row, row, row, full, vec,
                  pl.BlockSpec((_ROWBLK, 1), lambda i: (i, 0)),
                  full, vec, full, vec],
        out_specs=pl.BlockSpec((_G, _D), lambda i: (0, 0)),
        out_shape=jax.ShapeDtypeStruct((_G, _D), jnp.float32),
        scratch_shapes=[pltpu.VMEM((_G, _D), jnp.float32),
                        pltpu.VMEM((_G, _D), jnp.float32)],
    )(h, p0, p1, ws, bs, batch2d, w1, b1, w2p, b2p)


def kernel(x, edge_index, batch, Wl0, bl0, Ws0, bs0, Wl1, bl1, Ws1, bs1,
           Wl2, bl2, Ws2, bs2, g0, be0, g1, be1, W1, b1, W2, b2):
    src = edge_index[0]
    dst = edge_index[1]
    e = src.shape[0]
    chunks = -(-e // _CHUNK)
    chunks = -(-chunks // _NSUB) * _NSUB
    pad = chunks * _CHUNK - e
    src2d = jnp.concatenate([src, jnp.zeros((pad,), jnp.int32)]).reshape(
        chunks, _CHUNK)
    dst2d = jnp.concatenate([dst, jnp.zeros((pad,), jnp.int32)]).reshape(
        chunks, _CHUNK)
    zeros_init = jnp.zeros((_ZROWS, _D), jnp.float32)
    batch2d = batch.reshape(_N, 1)

    r = lambda v: v.reshape(1, _D)
    w2p = jnp.concatenate([W2, jnp.zeros((_D, _D - _NCLS), jnp.float32)], axis=1)
    b2p = jnp.concatenate([b2, jnp.zeros((_D - _NCLS,), jnp.float32)]).reshape(1, _D)

    xl0 = _first_lin(x, Wl0, r(bl0))
    parts = _sc_propagate(xl0, src2d, dst2d, zeros_init)
    h1, xl1 = _mid_layer(x, parts[:_N], parts[_ACC_ROWS:_ACC_ROWS + _N], Ws0,
                         r(bs0), r(g0), r(be0), Wl1, r(bl1))
    parts = _sc_propagate(xl1, src2d, dst2d, zeros_init)
    h2, xl2 = _mid_layer(h1, parts[:_N], parts[_ACC_ROWS:_ACC_ROWS + _N], Ws1,
                         r(bs1), r(g1), r(be1), Wl2, r(bl2))
    parts = _sc_propagate(xl2, src2d, dst2d, zeros_init)
    out = _final_stage(h2, parts[:_N], parts[_ACC_ROWS:_ACC_ROWS + _N], Ws2,
                       r(bs2), batch2d, W1, r(b1), w2p, b2p)
    return out[:, :_NCLS]
